# Initial kernel scaffold; baseline (speedup 1.0000x reference)
import jax
import jax.numpy as jnp
from jax.experimental import pallas as pl


def kernel(t, x, edge_index, W1, b1, W2, b2, W3, b3, W4, b4, W5, b5):
    raise NotImplementedError("write your pallas kernel here")


# speedup vs baseline: 7.4958x; 7.4958x over previous
import functools

import jax
import jax.numpy as jnp
from jax import lax
from jax.experimental import pallas as pl
from jax.experimental.pallas import tpu as pltpu
from jax.experimental.pallas import tpu_sc as plsc

N = 10000
NP = 10240
E = 320000
NTILES = 16
NCORES = 2
CHUNK = 128
ROWS_PT = NP // NTILES
F2 = 128

CA = -(-E // (NTILES * CHUNK))
EH = E // 2
CB = -(-EH // (NTILES * CHUNK))
EH_PAD = NTILES * CHUNK * CB

_MESH = plsc.VectorSubcoreMesh(core_axis_name="c", subcore_axis_name="s")


def _spmm_a_body(ys_hbm, src_hbm, dst_hbm, out_hbm, srcv, dstv, rows, out_sp):
    c = lax.axis_index("c")
    s = lax.axis_index("s")
    base = c * NP + s * ROWS_PT
    pltpu.sync_copy(ys_hbm.at[pl.ds(base, ROWS_PT)],
                    out_sp.at[pl.ds(s * ROWS_PT, ROWS_PT)])
    plsc.subcore_barrier()

    def body(j, carry):
        pltpu.sync_copy(src_hbm.at[c, s, j], srcv)
        pltpu.sync_copy(dst_hbm.at[s, j], dstv)
        pltpu.sync_copy(ys_hbm.at[srcv], rows)
        pltpu.sync_copy(rows, out_sp.at[dstv], add=True)
        return carry

    lax.fori_loop(0, CA, body, 0)
    plsc.subcore_barrier()
    pltpu.sync_copy(out_sp.at[pl.ds(s * ROWS_PT, ROWS_PT)],
                    out_hbm.at[pl.ds(base, ROWS_PT)])


def _spmm_b_body(ys_hbm, src_hbm, dst_hbm, zeros_hbm, out_hbm,
                 srcv, dstv, rows, out_sp):
    c = lax.axis_index("c")
    s = lax.axis_index("s")

    @pl.when(c == 0)
    def _():
        pltpu.sync_copy(ys_hbm.at[pl.ds(s * ROWS_PT, ROWS_PT)],
                        out_sp.at[pl.ds(s * ROWS_PT, ROWS_PT)])

    @pl.when(c != 0)
    def _():
        pltpu.sync_copy(zeros_hbm, out_sp.at[pl.ds(s * ROWS_PT, ROWS_PT)])

    plsc.subcore_barrier()

    def body(j, carry):
        pltpu.sync_copy(src_hbm.at[c, s, j], srcv)
        pltpu.sync_copy(dst_hbm.at[c, s, j], dstv)
        pltpu.sync_copy(ys_hbm.at[srcv], rows)
        pltpu.sync_copy(rows, out_sp.at[dstv], add=True)
        return carry

    lax.fori_loop(0, CB, body, 0)
    plsc.subcore_barrier()
    pltpu.sync_copy(out_sp.at[pl.ds(s * ROWS_PT, ROWS_PT)],
                    out_hbm.at[c, pl.ds(s * ROWS_PT, ROWS_PT)])


_spmm_a = pl.kernel(
    _spmm_a_body,
    out_type=jax.ShapeDtypeStruct((NCORES * NP, F2), jnp.float32),
    mesh=_MESH,
    scratch_types=[
        pltpu.VMEM((CHUNK,), jnp.int32),
        pltpu.VMEM((CHUNK,), jnp.int32),
        pltpu.VMEM((CHUNK, F2), jnp.float32),
        pltpu.VMEM_SHARED((NP, F2), jnp.float32),
    ],
)

_spmm_b = pl.kernel(
    _spmm_b_body,
    out_type=jax.ShapeDtypeStruct((NCORES, NP, F2), jnp.float32),
    mesh=_MESH,
    scratch_types=[
        pltpu.VMEM((CHUNK,), jnp.int32),
        pltpu.VMEM((CHUNK,), jnp.int32),
        pltpu.VMEM((CHUNK, F2), jnp.float32),
        pltpu.VMEM_SHARED((NP, F2), jnp.float32),
    ],
)


_BM = 1024


def _pad128(y):
    f = y.shape[1]
    if f == F2:
        return y
    return jnp.concatenate([y, jnp.zeros((y.shape[0], F2 - f), y.dtype)], 1)


def _mm_first_body(x_ref, deg_ref, w_ref, o_ref):
    dinv = lax.rsqrt(deg_ref[...])
    y = jnp.dot(x_ref[...] * dinv, w_ref[...],
                preferred_element_type=jnp.float32,
                precision=lax.Precision.HIGHEST)
    o_ref[...] = _pad128(y)


def _mm_mid_body(in_mode, fin, out_mode, sy_ref, deg_ref, b_ref, w_ref, o_ref):
    dinv = lax.rsqrt(deg_ref[...])
    if in_mode == "add":
        sfull = (sy_ref[0] + sy_ref[1])[:, :fin]
    else:
        sfull = jnp.concatenate([sy_ref[0], sy_ref[1]], axis=1)
    h = jnp.tanh(sfull * dinv + b_ref[...])
    y = jnp.dot(h * dinv, w_ref[...],
                preferred_element_type=jnp.float32,
                precision=lax.Precision.HIGHEST)
    if out_mode == "split":
        f2 = y.shape[1] // 2
        o_ref[0] = y[:, :f2]
        o_ref[1] = y[:, f2:]
    else:
        o_ref[...] = _pad128(y)


def _mm_final_body(sy_ref, deg_ref, b_ref, o_ref):
    dinv = lax.rsqrt(deg_ref[...])
    sfull = sy_ref[0] + sy_ref[1]
    o_ref[...] = sfull * dinv + b_ref[...]


def _out_spec(out_mode, fo):
    if out_mode == "split":
        return (pl.BlockSpec((2, _BM, fo // 2), lambda i: (0, i, 0)),
                jax.ShapeDtypeStruct((2, NP, fo // 2), jnp.float32))
    return (pl.BlockSpec((_BM, F2), lambda i: (i, 0)),
            jax.ShapeDtypeStruct((NP, F2), jnp.float32))


def _mm_first(x, deg, w):
    ospec, oshape = _out_spec("plain", F2)
    return pl.pallas_call(
        _mm_first_body,
        grid=(NP // _BM,),
        in_specs=[
            pl.BlockSpec((_BM, x.shape[1]), lambda i: (i, 0)),
            pl.BlockSpec((_BM, 1), lambda i: (i, 0)),
            pl.BlockSpec(w.shape, lambda i: (0, 0)),
        ],
        out_specs=ospec,
        out_shape=oshape,
    )(x, deg, w)


def _mm_mid(in_mode, fin, out_mode, sy, deg, b, w):
    fo = w.shape[1]
    fp2 = sy.shape[2]
    ospec, oshape = _out_spec(out_mode, fo)
    return pl.pallas_call(
        functools.partial(_mm_mid_body, in_mode, fin, out_mode),
        grid=(NP // _BM,),
        in_specs=[
            pl.BlockSpec((2, _BM, fp2), lambda i: (0, i, 0)),
            pl.BlockSpec((_BM, 1), lambda i: (i, 0)),
            pl.BlockSpec(b.shape, lambda i: (0, 0)),
            pl.BlockSpec(w.shape, lambda i: (0, 0)),
        ],
        out_specs=ospec,
        out_shape=oshape,
    )(sy, deg, b, w)


def _mm_final(sy, deg, b):
    return pl.pallas_call(
        _mm_final_body,
        grid=(NP // _BM,),
        in_specs=[
            pl.BlockSpec((2, _BM, F2), lambda i: (0, i, 0)),
            pl.BlockSpec((_BM, 1), lambda i: (i, 0)),
            pl.BlockSpec(b.shape, lambda i: (0, 0)),
        ],
        out_specs=pl.BlockSpec((_BM, F2), lambda i: (i, 0)),
        out_shape=jax.ShapeDtypeStruct((NP, F2), jnp.float32),
    )(sy, deg, b)


def kernel(t, x, edge_index, W1, b1, W2, b2, W3, b3, W4, b4, W5, b5):
    src = edge_index[0]
    dst = edge_index[1]

    padv = jnp.full((NTILES * CHUNK * CA - E,), N, dtype=jnp.int32)
    src_r = jnp.concatenate([src, padv]).reshape(NTILES, CA, CHUNK)
    dst_a = jnp.concatenate([dst, padv]).reshape(NTILES, CA, CHUNK)
    src_a = jnp.stack([src_r, src_r + NP])

    def split_b(v):
        out = jnp.full((NCORES, EH_PAD), N, dtype=jnp.int32)
        out = out.at[:, :EH].set(v.reshape(NCORES, EH))
        return out.reshape(NCORES, NTILES, CB, CHUNK)

    src_b = split_b(src)
    dst_b = split_b(dst)

    zeros = jnp.zeros((ROWS_PT, F2), jnp.float32)
    ones = jnp.ones((NP, F2), jnp.float32)

    dsum = _spmm_b(ones, src_b, dst_b, zeros)
    deg = dsum[0, :, :1] + dsum[1, :, :1]

    xp = jnp.pad(x, ((0, NP - N), (0, 0)))

    y1 = _mm_first(xp, deg, W1)
    sy1 = _spmm_b(y1, src_b, dst_b, zeros)
    y2 = _mm_mid("add", 64, "split", sy1, deg, b1.reshape(1, -1), W2)
    sy2 = _spmm_a(y2.reshape(2 * NP, F2), src_a, dst_a).reshape(2, NP, F2)
    y3 = _mm_mid("concat", 256, "split", sy2, deg, b2.reshape(1, -1), W3)
    sy3 = _spmm_a(y3.reshape(2 * NP, F2), src_a, dst_a).reshape(2, NP, F2)
    y4 = _mm_mid("concat", 256, "plain", sy3, deg, b3.reshape(1, -1), W4)
    sy4 = _spmm_b(y4, src_b, dst_b, zeros)
    y5 = _mm_mid("add", 64, "plain", sy4, deg, b4.reshape(1, -1), W5)
    sy5 = _spmm_b(y5, src_b, dst_b, zeros)
    out = _mm_final(sy5, deg, b5.reshape(1, -1))
    return out[:N]
